# Initial kernel scaffold; baseline (speedup 1.0000x reference)
import jax
import jax.numpy as jnp
from jax.experimental import pallas as pl


def kernel(r_index_sq, r_val_sq, t_index_sq, adj_idx, ent_sparse, rel_sparse, time_sparse, ent_table, rel_table, time_table, attend_rel_w, attend_rel_b, attend_time_w, attend_time_b):
    raise NotImplementedError("write your pallas kernel here")


# speedup vs baseline: 13.0398x; 13.0398x over previous
import jax
import jax.numpy as jnp
from jax import lax
from jax.experimental import pallas as pl
from jax.experimental.pallas import tpu as pltpu
from jax.experimental.pallas import tpu_sc as plsc

NODE = 10000
REL = 500
TRIPLE = 160000
EMB = 128

NC = 2
NS = 16
L = 16

NP = 10240
RP = 512
EP = 40960
SP = 4096
TP = 163840
WF = 512 * 512

F32 = jnp.float32
I32 = jnp.int32

_SC_PARAMS = pltpu.CompilerParams(needs_layout_passes=False)


def _mesh():
    return plsc.VectorSubcoreMesh(core_axis_name="c", subcore_axis_name="s",
                                  num_cores=NC, num_subcores=NS)


def _zero_gbuf(gbuf):
    def _z(i, _):
        gbuf[i // 8, pl.ds((i % 8) * L, L)] = jnp.zeros((L,), F32)
        return 0
    lax.fori_loop(0, 1024, _z, 0)


def _zero_1d(buf, n):
    def _z(i, _):
        buf[pl.ds(i * L, L)] = jnp.zeros((L,), F32)
        return 0
    lax.fori_loop(0, n // L, _z, 0)


def _sc_ent(ent_src, ent_dst, ent_tab,
            eacc_o,
            idxa, idxb, ones_b, gbuf, zflat, rcpb, sem,
            s_eacc, s_ecnt):
    cid = lax.axis_index("c")
    sid = lax.axis_index("s")
    wid = sid * NC + cid

    _zero_gbuf(gbuf)
    _zero_1d(zflat, 640)
    for i in range(8):
        ones_b[0, pl.ds(i * L, L)] = jnp.ones((L,), F32)
    for k in range(5):
        pltpu.sync_copy(gbuf, s_eacc.at[pl.ds(sid * 640 + k * 128, 128)])
    pltpu.sync_copy(zflat, s_ecnt.at[pl.ds(sid * 640, 640)])
    plsc.subcore_barrier()

    pltpu.sync_copy(ent_dst.at[pl.ds(sid * 20, 20)], idxb)
    pltpu.sync_copy(ent_src.at[pl.ds(wid * 10, 10)], idxa)
    for j in range(20):
        pltpu.sync_copy(ones_b.at[0], s_ecnt.at[idxb.at[j, 0]], add=True)

    def _ent(j, _):
        pltpu.async_copy(ent_tab.at[idxa.at[j, 0]], gbuf, sem).wait()
        pltpu.sync_copy(gbuf, s_eacc.at[idxb.at[cid * 10 + j, 0]], add=True)
        return 0
    lax.fori_loop(0, 10, _ent, 0)

    plsc.subcore_barrier()

    pltpu.sync_copy(s_ecnt.at[pl.ds(sid * 640, 640)], rcpb)

    def _rcp(i, _):
        sl = pl.ds(i * L, L)
        rcpb[sl] = 1.0 / jnp.maximum(rcpb[sl], 1.0)
        return 0
    lax.fori_loop(0, 40, _rcp, 0)

    for k in range(5):
        lo = sid * 640 + k * 128
        pltpu.sync_copy(s_eacc.at[pl.ds(lo, 128)], gbuf)

        def _scale(r, _):
            sv = plsc.load_gather(rcpb, [jnp.full((L,), k * 128 + r, I32)])
            for f in range(8):
                sl = pl.ds(f * L, L)
                gbuf[r, sl] = gbuf[r, sl] * sv
            return 0
        lax.fori_loop(0, 128, _scale, 0)
        pltpu.sync_copy(gbuf, eacc_o.at[cid, pl.ds(pl.multiple_of(lo, 64), 128)])


def _run_sc_ent(ent_src, ent_dst, ent_tab):
    scratch = [
        pltpu.VMEM((10, 1, 128), I32),
        pltpu.VMEM((20, 1, 128), I32),
        pltpu.VMEM((1, 128), F32),
        pltpu.VMEM((128, 128), F32),
        pltpu.VMEM((640,), F32),
        pltpu.VMEM((640,), F32),
        pltpu.SemaphoreType.DMA,
        pltpu.VMEM_SHARED((NP, EMB), F32),
        pltpu.VMEM_SHARED((NP,), F32),
    ]
    fn = pl.kernel(_sc_ent, out_type=jax.ShapeDtypeStruct((NC, NP, EMB), F32),
                   mesh=_mesh(), scratch_types=scratch,
                   compiler_params=_SC_PARAMS)
    return fn(ent_src, ent_dst, ent_tab)


def _sc_relw(rel_src, rel_dst, tim_src, tim_dst,
             trr, trc, ttr, ttc, tval, rel_tab, tim_tab,
             racc_o, tacc_o, wr_o, wt_o,
             ridxa, ridxb, tidxa, tidxb,
             rr_b, rc_b, tr_b, tc_b, tv_b, tflat, ones_b,
             gbuf, zflat, dbuf, rcps, sem,
             s_racc, s_rcnt, s_tacc, s_tcnt, s_wr, s_wt):
    cid = lax.axis_index("c")
    sid = lax.axis_index("s")
    wid = sid * NC + cid

    _zero_gbuf(gbuf)
    _zero_1d(zflat, 2048)
    for i in range(8):
        ones_b[0, pl.ds(i * L, L)] = jnp.ones((L,), F32)
    pltpu.sync_copy(gbuf.at[pl.ds(0, 32)], s_racc.at[pl.ds(sid * 32, 32)])
    pltpu.sync_copy(gbuf.at[pl.ds(0, 32)], s_tacc.at[pl.ds(sid * 32, 32)])
    pltpu.sync_copy(zflat.at[pl.ds(0, 32)], s_rcnt.at[pl.ds(sid * 32, 32)])
    pltpu.sync_copy(zflat.at[pl.ds(0, 32)], s_tcnt.at[pl.ds(sid * 32, 32)])
    for k in range(8):
        pltpu.sync_copy(zflat, s_wr.at[pl.ds(sid * 16384 + k * 2048, 2048)])
        pltpu.sync_copy(zflat, s_wt.at[pl.ds(sid * 16384 + k * 2048, 2048)])
    plsc.subcore_barrier()

    pltpu.sync_copy(rel_dst.at[pl.ds(sid * 2, 2)], ridxb)
    pltpu.sync_copy(rel_src.at[pl.ds(wid, 1)], ridxa)
    pltpu.sync_copy(tim_dst.at[pl.ds(sid * 2, 2)], tidxb)
    pltpu.sync_copy(tim_src.at[pl.ds(wid, 1)], tidxa)
    for j in range(2):
        pltpu.sync_copy(ones_b.at[0], s_rcnt.at[ridxb.at[j, 0]], add=True)
        pltpu.sync_copy(ones_b.at[0], s_tcnt.at[tidxb.at[j, 0]], add=True)
    pltpu.async_copy(rel_tab.at[ridxa.at[0, 0]], gbuf, sem).wait()
    pltpu.sync_copy(gbuf, s_racc.at[ridxb.at[cid, 0]], add=True)
    pltpu.async_copy(tim_tab.at[tidxa.at[0, 0]], gbuf, sem).wait()
    pltpu.sync_copy(gbuf, s_tacc.at[tidxb.at[cid, 0]], add=True)

    pltpu.sync_copy(trr.at[pl.ds(wid * 40, 40)], rr_b)
    pltpu.sync_copy(trc.at[pl.ds(wid * 40, 40)], rc_b)
    pltpu.sync_copy(ttr.at[pl.ds(wid * 40, 40)], tr_b)
    pltpu.sync_copy(ttc.at[pl.ds(wid * 40, 40)], tc_b)
    pltpu.sync_copy(tval.at[pl.ds(wid * 40, 40)], tv_b)

    def _tri(j, _):
        for jj in range(8):
            sl = pl.ds(jj * L, L)
            tflat[0, sl] = rr_b[j, 0, sl] * 512 + rc_b[j, 0, sl]
        pltpu.sync_copy(tv_b.at[j, 0], s_wr.at[tflat.at[0]], add=True)
        for jj in range(8):
            sl = pl.ds(jj * L, L)
            tflat[0, sl] = tr_b[j, 0, sl] * 512 + tc_b[j, 0, sl]
        pltpu.sync_copy(tv_b.at[j, 0], s_wt.at[tflat.at[0]], add=True)
        return 0
    lax.fori_loop(0, 40, _tri, 0)

    plsc.subcore_barrier()

    for (cnt_ref, acc_ref, out_ref) in ((s_rcnt, s_racc, racc_o),
                                        (s_tcnt, s_tacc, tacc_o)):
        pltpu.sync_copy(cnt_ref.at[pl.ds(sid * 32, 32)], rcps.at[pl.ds(0, 32)])
        for i in range(2):
            sl = pl.ds(i * L, L)
            rcps[sl] = 1.0 / jnp.maximum(rcps[sl], 1.0)
        pltpu.sync_copy(acc_ref.at[pl.ds(sid * 32, 32)], dbuf)

        def _scale_s(r, _):
            sv = plsc.load_gather(rcps, [jnp.full((L,), r, I32)])
            for f in range(8):
                sl = pl.ds(f * L, L)
                dbuf[r, sl] = dbuf[r, sl] * sv
            return 0
        lax.fori_loop(0, 32, _scale_s, 0)
        pltpu.sync_copy(dbuf,
                        out_ref.at[cid, pl.ds(pl.multiple_of(sid * 32, 8), 32)])

    for k in range(8):
        o = sid * 16384 + k * 2048
        pltpu.sync_copy(s_wr.at[pl.ds(o, 2048)], zflat)
        pltpu.sync_copy(zflat, wr_o.at[cid, pl.ds(pl.multiple_of(o, 128), 2048)])
        pltpu.sync_copy(s_wt.at[pl.ds(o, 2048)], zflat)
        pltpu.sync_copy(zflat, wt_o.at[cid, pl.ds(pl.multiple_of(o, 128), 2048)])


def _run_sc_relw(rel_src, rel_dst, tim_src, tim_dst,
                 trr, trc, ttr, ttc, tval, rel_tab, tim_tab):
    out_type = (
        jax.ShapeDtypeStruct((NC, RP, EMB), F32),
        jax.ShapeDtypeStruct((NC, RP, EMB), F32),
        jax.ShapeDtypeStruct((NC, WF), F32),
        jax.ShapeDtypeStruct((NC, WF), F32),
    )
    scratch = [
        pltpu.VMEM((1, 1, 128), I32),
        pltpu.VMEM((2, 1, 128), I32),
        pltpu.VMEM((1, 1, 128), I32),
        pltpu.VMEM((2, 1, 128), I32),
        pltpu.VMEM((40, 1, 128), I32),
        pltpu.VMEM((40, 1, 128), I32),
        pltpu.VMEM((40, 1, 128), I32),
        pltpu.VMEM((40, 1, 128), I32),
        pltpu.VMEM((40, 1, 128), F32),
        pltpu.VMEM((1, 128), I32),
        pltpu.VMEM((1, 128), F32),
        pltpu.VMEM((128, 128), F32),
        pltpu.VMEM((2048,), F32),
        pltpu.VMEM((32, 128), F32),
        pltpu.VMEM((64,), F32),
        pltpu.SemaphoreType.DMA,
        pltpu.VMEM_SHARED((RP, EMB), F32),
        pltpu.VMEM_SHARED((RP,), F32),
        pltpu.VMEM_SHARED((RP, EMB), F32),
        pltpu.VMEM_SHARED((RP,), F32),
        pltpu.VMEM_SHARED((WF,), F32),
        pltpu.VMEM_SHARED((WF,), F32),
    ]
    fn = pl.kernel(_sc_relw, out_type=out_type, mesh=_mesh(),
                   scratch_types=scratch, compiler_params=_SC_PARAMS)
    return fn(rel_src, rel_dst, tim_src, tim_dst,
              trr, trc, ttr, ttc, tval, rel_tab, tim_tab)


def _tc_small(racc, tacc, wr2, wt2, wcr, wct,
              cr_o, ct_o, u_o, mcr_o, mct_o):
    rel_emb = racc[0] + racc[1]
    tim_emb = tacc[0] + tacc[1]
    w_r = wr2[0] + wr2[1]
    w_t = wt2[0] + wt2[1]
    rels = jnp.dot(w_r, rel_emb, preferred_element_type=F32)
    times = jnp.dot(w_t, tim_emb, preferred_element_type=F32)
    u_o[...] = times
    rn = rels / jnp.maximum(
        jnp.sqrt(jnp.sum(rels * rels, axis=1, keepdims=True)), 1e-12)
    tn = times / jnp.maximum(
        jnp.sqrt(jnp.sum(times * times, axis=1, keepdims=True)), 1e-12)
    cr = jnp.dot(rn, wcr[...], preferred_element_type=F32)
    ct = jnp.dot(tn, wct[...], preferred_element_type=F32)
    cr_o[...] = cr
    ct_o[...] = ct
    mcr_o[...] = jnp.broadcast_to(jnp.max(cr), (1, 128))
    mct_o[...] = jnp.broadcast_to(jnp.max(ct), (1, 128))


def _run_tc_small(racc, tacc, wr2, wt2, wcr, wct):
    return pl.pallas_call(
        _tc_small,
        out_shape=(
            jax.ShapeDtypeStruct((RP, 1), F32),
            jax.ShapeDtypeStruct((RP, 1), F32),
            jax.ShapeDtypeStruct((RP, EMB), F32),
            jax.ShapeDtypeStruct((1, 128), F32),
            jax.ShapeDtypeStruct((1, 128), F32),
        ),
    )(racc, tacc, wr2, wt2, wcr, wct)


def _tc_proj(e0, e1, wcols, ent_o, relu_o, pq_o, mx_o):
    i = pl.program_id(0)
    ent = e0[...] + e1[...]
    ent_o[...] = ent
    relu_o[...] = jnp.maximum(ent, 0.0)
    pq = jnp.dot(ent, wcols[...], preferred_element_type=F32)
    pq_o[...] = pq
    bm = jnp.max(pq, axis=0, keepdims=True)

    @pl.when(i == 0)
    def _():
        mx_o[...] = jnp.full((8, 8), -3e38, F32)
    mx_o[0:1, :] = jnp.maximum(mx_o[0:1, :], bm)


def _run_tc_proj(e0, e1, wcols):
    nb = NP // 1024
    return pl.pallas_call(
        _tc_proj,
        grid=(nb,),
        in_specs=[
            pl.BlockSpec((1024, EMB), lambda i: (i, 0)),
            pl.BlockSpec((1024, EMB), lambda i: (i, 0)),
            pl.BlockSpec((EMB, 8), lambda i: (0, 0)),
        ],
        out_specs=(
            pl.BlockSpec((1024, EMB), lambda i: (i, 0)),
            pl.BlockSpec((1024, EMB), lambda i: (i, 0)),
            pl.BlockSpec((1024, 8), lambda i: (i, 0)),
            pl.BlockSpec((8, 8), lambda i: (0, 0)),
        ),
        out_shape=(
            jax.ShapeDtypeStruct((NP, EMB), F32),
            jax.ShapeDtypeStruct((NP, EMB), F32),
            jax.ShapeDtypeStruct((NP, 8), F32),
            jax.ShapeDtypeStruct((8, 8), F32),
        ),
    )(e0, e1, wcols)


def _sc_soft(rows3d, cols3d, pqt, crow, biasv, lane,
             a_o,
             prb, qrb, ptb, qtb, crb, ctb, bb, lb,
             rbuf, cbuf, sbr, sbt, zflat,
             s_dr, s_dt):
    cid = lax.axis_index("c")
    sid = lax.axis_index("s")
    wid = sid * NC + cid

    _zero_1d(zflat, 640)
    pltpu.sync_copy(zflat, s_dr.at[pl.ds(sid * 640, 640)])
    pltpu.sync_copy(zflat, s_dt.at[pl.ds(sid * 640, 640)])

    pltpu.sync_copy(pqt.at[0, 0], prb)
    pltpu.sync_copy(pqt.at[1, 0], qrb)
    pltpu.sync_copy(pqt.at[2, 0], ptb)
    pltpu.sync_copy(pqt.at[3, 0], qtb)
    pltpu.sync_copy(crow.at[0, 0], crb)
    pltpu.sync_copy(crow.at[1, 0], ctb)
    pltpu.sync_copy(biasv, bb)
    pltpu.sync_copy(lane, lb)
    pltpu.sync_copy(rows3d.at[pl.ds(sid * 80, 80)], rbuf)
    pltpu.sync_copy(cols3d.at[pl.ds(sid * 80, 80)], cbuf)
    plsc.subcore_barrier()

    bias_r = bb[pl.ds(0, L)]
    bias_t = bb[pl.ds(L, L)]
    lanev = lb[...]

    def _p1(j, _):
        for jj in range(8):
            sl = pl.ds(jj * L, L)
            r16 = rbuf[j, 0, sl]
            c16 = cbuf[j, 0, sl]
            e16 = sid * 10240 + j * 128 + jj * L + lanev
            cidx = jnp.minimum(e16, 511)
            xr = (plsc.load_gather(prb, [r16]) + plsc.load_gather(qrb, [c16])
                  + plsc.load_gather(crb, [cidx]) + bias_r)
            xt = (plsc.load_gather(ptb, [r16]) + plsc.load_gather(qtb, [c16])
                  + plsc.load_gather(ctb, [cidx]) + bias_t)
            sbr[j, 0, sl] = jnp.exp(xr)
            sbt[j, 0, sl] = jnp.exp(xt)
        pltpu.sync_copy(sbr.at[j, 0], s_dr.at[rbuf.at[j, 0]], add=True)
        pltpu.sync_copy(sbt.at[j, 0], s_dt.at[rbuf.at[j, 0]], add=True)
        return 0
    lax.fori_loop(0, 80, _p1, 0)

    plsc.subcore_barrier()

    pltpu.sync_copy(s_dr, prb)
    pltpu.sync_copy(s_dt, qrb)

    def _p2(j2, _):
        j = cid * 40 + j2
        for jj in range(8):
            sl = pl.ds(jj * L, L)
            r16 = rbuf[j, 0, sl]
            dr = plsc.load_gather(prb, [r16])
            dt = plsc.load_gather(qrb, [r16])
            sbr[j, 0, sl] = (sbr[j, 0, sl] / jnp.maximum(dr, 1e-12)
                             + sbt[j, 0, sl] / jnp.maximum(dt, 1e-12))
        pltpu.sync_copy(sbr.at[j, 0],
                        a_o.at[sid * 80 + j, 0])
        return 0
    lax.fori_loop(0, 40, _p2, 0)


def _run_sc_soft(rows3d, cols3d, pqt, crow, biasv, lane):
    scratch = [
        pltpu.VMEM((NP,), F32),
        pltpu.VMEM((NP,), F32),
        pltpu.VMEM((NP,), F32),
        pltpu.VMEM((NP,), F32),
        pltpu.VMEM((RP,), F32),
        pltpu.VMEM((RP,), F32),
        pltpu.VMEM((32,), F32),
        pltpu.VMEM((16,), I32),
        pltpu.VMEM((80, 1, 128), I32),
        pltpu.VMEM((80, 1, 128), I32),
        pltpu.VMEM((80, 1, 128), F32),
        pltpu.VMEM((80, 1, 128), F32),
        pltpu.VMEM((640,), F32),
        pltpu.VMEM_SHARED((NP,), F32),
        pltpu.VMEM_SHARED((NP,), F32),
    ]
    fn = pl.kernel(_sc_soft,
                   out_type=jax.ShapeDtypeStruct((TP // 128, 1, 128), F32),
                   mesh=_mesh(), scratch_types=scratch,
                   compiler_params=_SC_PARAMS)
    return fn(rows3d, cols3d, pqt, crow, biasv, lane)


def _sc_agg(rows3d, cols3d, a3, ent_emb,
            acc_o,
            rbuf, cbuf, abuf, gbuf, sem,
            s_acc):
    cid = lax.axis_index("c")
    sid = lax.axis_index("s")
    wid = sid * NC + cid

    _zero_gbuf(gbuf)
    for k in range(5):
        pltpu.sync_copy(gbuf, s_acc.at[pl.ds(sid * 640 + k * 128, 128)])
    pltpu.sync_copy(rows3d.at[pl.ds(wid * 40, 40)], rbuf)
    pltpu.sync_copy(cols3d.at[pl.ds(wid * 40, 40)], cbuf)
    pltpu.sync_copy(a3.at[pl.ds(wid * 40, 40)], abuf)
    plsc.subcore_barrier()

    def _p2(j, _):
        pltpu.async_copy(ent_emb.at[cbuf.at[j, 0]], gbuf, sem).wait()

        def _scale(r, _):
            jf = jnp.full((L,), j, I32)
            av = plsc.load_gather(abuf, [jf, jnp.minimum(jf, 0),
                                         jnp.full((L,), r, I32)])
            for f in range(8):
                sl = pl.ds(f * L, L)
                gbuf[r, sl] = gbuf[r, sl] * av
            return 0
        lax.fori_loop(0, 128, _scale, 0)
        pltpu.sync_copy(gbuf, s_acc.at[rbuf.at[j, 0]], add=True)
        return 0
    lax.fori_loop(0, 40, _p2, 0)

    plsc.subcore_barrier()
    for k in range(5):
        lo = sid * 640 + k * 128
        pltpu.sync_copy(s_acc.at[pl.ds(lo, 128)], gbuf)
        pltpu.sync_copy(gbuf, acc_o.at[cid, pl.ds(pl.multiple_of(lo, 64), 128)])


def _run_sc_agg(rows3d, cols3d, a3, ent_emb):
    scratch = [
        pltpu.VMEM((40, 1, 128), I32),
        pltpu.VMEM((40, 1, 128), I32),
        pltpu.VMEM((40, 1, 128), F32),
        pltpu.VMEM((128, 128), F32),
        pltpu.SemaphoreType.DMA,
        pltpu.VMEM_SHARED((NP, EMB), F32),
    ]
    fn = pl.kernel(_sc_agg,
                   out_type=jax.ShapeDtypeStruct((NC, NP, EMB), F32),
                   mesh=_mesh(), scratch_types=scratch,
                   compiler_params=_SC_PARAMS)
    return fn(rows3d, cols3d, a3, ent_emb)


def _tc_out(n0, n1, relu_ent, u, rows512, out_o):
    i = pl.program_id(0)
    feat = jnp.maximum(n0[...] + n1[...], 0.0)
    node_ids = lax.broadcasted_iota(I32, (RP, 1000), 1) + i * 1000
    onehot = jnp.where(node_ids == rows512[:, 0:1], 1.0, 0.0)
    tenc = jnp.maximum(
        lax.dot_general(onehot, u[...], (((0,), (0,)), ((), ())),
                        preferred_element_type=F32), 0.0)
    out_o[:, 0:128] = relu_ent[...]
    out_o[:, 128:256] = feat
    out_o[:, 256:384] = feat
    out_o[:, 384:512] = tenc


def _run_tc_out(n0, n1, relu_ent, u, rows512):
    return pl.pallas_call(
        _tc_out,
        grid=(10,),
        in_specs=[
            pl.BlockSpec((1000, EMB), lambda i: (i, 0)),
            pl.BlockSpec((1000, EMB), lambda i: (i, 0)),
            pl.BlockSpec((1000, EMB), lambda i: (i, 0)),
            pl.BlockSpec((RP, EMB), lambda i: (0, 0)),
            pl.BlockSpec((RP, 8), lambda i: (0, 0)),
        ],
        out_specs=pl.BlockSpec((1000, 4 * EMB), lambda i: (i, 0)),
        out_shape=jax.ShapeDtypeStruct((NODE, 4 * EMB), F32),
    )(n0, n1, relu_ent, u, rows512)


def kernel(r_index_sq, r_val_sq, t_index_sq, adj_idx, ent_sparse, rel_sparse,
           time_sparse, ent_table, rel_table, time_table, attend_rel_w,
           attend_rel_b, attend_time_w, attend_time_b):
    def pad1(x, n, fill):
        x = x.astype(I32)
        return jnp.concatenate([x, fill.astype(I32)]) if n > x.shape[0] else x

    dump16 = lambda m: NODE + (jnp.arange(m, dtype=I32) % 16)

    ent_src = pad1(ent_sparse[1], EP, jnp.full((EP - 4 * NODE,), NODE + 8))
    ent_dst = pad1(ent_sparse[0], EP, dump16(EP - 4 * NODE))
    rel_src = pad1(rel_sparse[1], SP, jnp.full((SP - 8 * REL,), NODE + 8))
    rel_dst = pad1(rel_sparse[0], SP,
                   504 + (jnp.arange(SP - 8 * REL, dtype=I32) % 8))
    tim_src = pad1(time_sparse[1], SP, jnp.full((SP - 8 * REL,), NODE + 8))
    tim_dst = pad1(time_sparse[0], SP,
                   504 + (jnp.arange(SP - 8 * REL, dtype=I32) % 8))

    npad = TP - TRIPLE
    trr = pad1(r_index_sq[:, 0], TP, jnp.full((npad,), 500))
    trc = pad1(r_index_sq[:, 1], TP, jnp.arange(npad, dtype=I32) % 500)
    ttr = pad1(t_index_sq[:, 0], TP, jnp.full((npad,), 500))
    ttc = pad1(t_index_sq[:, 1], TP, jnp.arange(npad, dtype=I32) % 500)
    tval = jnp.concatenate([r_val_sq, jnp.zeros((npad,), F32)])

    rows = pad1(adj_idx[0], TP, dump16(npad))
    cols = pad1(adj_idx[1], TP, jnp.full((npad,), NODE + 8))

    pad_tab = lambda t: jnp.concatenate(
        [t, jnp.zeros((NP - NODE, EMB), F32)], axis=0)
    ent_tab = pad_tab(ent_table)
    rel_tab = pad_tab(rel_table)
    tim_tab = pad_tab(time_table)

    r3 = lambda x, n: x.reshape(n, 1, 128)

    eacc = _run_sc_ent(r3(ent_src, EP // 128), r3(ent_dst, EP // 128),
                       ent_tab)
    racc, tacc, wr, wt = _run_sc_relw(
        r3(rel_src, SP // 128), r3(rel_dst, SP // 128),
        r3(tim_src, SP // 128), r3(tim_dst, SP // 128),
        r3(trr, TP // 128), r3(trc, TP // 128),
        r3(ttr, TP // 128), r3(ttc, TP // 128),
        r3(tval, TP // 128), rel_tab, tim_tab)

    wcr = attend_rel_w[0, 256:384].reshape(EMB, 1)
    wct = attend_time_w[0, 256:384].reshape(EMB, 1)
    cr, ct, u, mcr, mct = _run_tc_small(
        racc, tacc, wr.reshape(NC, 512, 512), wt.reshape(NC, 512, 512),
        wcr, wct)

    wcols = jnp.stack([attend_rel_w[0, 0:128], attend_rel_w[0, 128:256],
                       attend_time_w[0, 0:128], attend_time_w[0, 128:256],
                       jnp.zeros((EMB,), F32), jnp.zeros((EMB,), F32),
                       jnp.zeros((EMB,), F32), jnp.zeros((EMB,), F32)],
                      axis=1)
    ent_emb, relu_ent, pq, mx = _run_tc_proj(eacc[0], eacc[1], wcols)

    pqt = pq.T[0:4].reshape(4, 1, NP)
    crow = jnp.stack([cr.reshape(RP), ct.reshape(RP)]).reshape(2, 1, RP)
    bias_r = attend_rel_b[0] - mx[0, 0] - mx[0, 1] - mcr[0, 0]
    bias_t = attend_time_b[0] - mx[0, 2] - mx[0, 3] - mct[0, 0]
    biasv = jnp.concatenate([jnp.full((16,), bias_r, F32),
                             jnp.full((16,), bias_t, F32)])

    a3 = _run_sc_soft(r3(rows, TP // 128), r3(cols, TP // 128),
                      pqt, crow, biasv, jnp.arange(16, dtype=I32))
    acc = _run_sc_agg(r3(rows, TP // 128), r3(cols, TP // 128), a3, ent_emb)

    rows512 = jnp.concatenate(
        [adj_idx[0][:500].astype(I32), jnp.full((12,), 1 << 28, I32)])
    rows512 = jnp.broadcast_to(rows512.reshape(RP, 1), (RP, 8))
    return _run_tc_out(acc[0], acc[1], relu_ent, u, rows512)


# speedup vs baseline: 14.7407x; 1.1304x over previous
import jax
import jax.numpy as jnp
from jax import lax
from jax.experimental import pallas as pl
from jax.experimental.pallas import tpu as pltpu
from jax.experimental.pallas import tpu_sc as plsc

NODE = 10000
REL = 500
TRIPLE = 160000
EMB = 128

NC = 2
NS = 16
L = 16

NP = 10240
RP = 512
EP = 40960
SP = 4096
TP = 163840
WF = 512 * 512

F32 = jnp.float32
I32 = jnp.int32

_SC_PARAMS = pltpu.CompilerParams(needs_layout_passes=False)


def _mesh():
    return plsc.VectorSubcoreMesh(core_axis_name="c", subcore_axis_name="s",
                                  num_cores=NC, num_subcores=NS)


def _zero_gbuf(gbuf):
    def _z(i, _):
        gbuf[i // 8, pl.ds((i % 8) * L, L)] = jnp.zeros((L,), F32)
        return 0
    lax.fori_loop(0, 1024, _z, 0)


def _zero_1d(buf, n):
    def _z(i, _):
        buf[pl.ds(i * L, L)] = jnp.zeros((L,), F32)
        return 0
    lax.fori_loop(0, n // L, _z, 0)


def _sc_ent(ent_src, ent_dst, ent_tab,
            eacc_o,
            idxa, idxb, ones_b, gbuf, zflat, rcpb, sem,
            s_eacc, s_ecnt):
    cid = lax.axis_index("c")
    sid = lax.axis_index("s")
    wid = sid * NC + cid

    _zero_gbuf(gbuf)
    _zero_1d(zflat, 640)
    for i in range(8):
        ones_b[0, pl.ds(i * L, L)] = jnp.ones((L,), F32)
    for k in range(5):
        pltpu.sync_copy(gbuf, s_eacc.at[pl.ds(sid * 640 + k * 128, 128)])
    pltpu.sync_copy(zflat, s_ecnt.at[pl.ds(sid * 640, 640)])
    plsc.subcore_barrier()

    pltpu.sync_copy(ent_dst.at[pl.ds(sid * 20, 20)], idxb)
    pltpu.sync_copy(ent_src.at[pl.ds(wid * 10, 10)], idxa)
    for j in range(20):
        pltpu.sync_copy(ones_b.at[0], s_ecnt.at[idxb.at[j, 0]], add=True)

    def _ent(j, _):
        pltpu.async_copy(ent_tab.at[idxa.at[j, 0]], gbuf, sem).wait()
        pltpu.sync_copy(gbuf, s_eacc.at[idxb.at[cid * 10 + j, 0]], add=True)
        return 0
    lax.fori_loop(0, 10, _ent, 0)

    plsc.subcore_barrier()

    pltpu.sync_copy(s_ecnt.at[pl.ds(sid * 640, 640)], rcpb)

    def _rcp(i, _):
        sl = pl.ds(i * L, L)
        rcpb[sl] = 1.0 / jnp.maximum(rcpb[sl], 1.0)
        return 0
    lax.fori_loop(0, 40, _rcp, 0)

    for k in range(5):
        lo = sid * 640 + k * 128
        pltpu.sync_copy(s_eacc.at[pl.ds(lo, 128)], gbuf)

        def _scale(r, _):
            sv = plsc.load_gather(rcpb, [jnp.full((L,), k * 128 + r, I32)])
            for f in range(8):
                sl = pl.ds(f * L, L)
                gbuf[r, sl] = gbuf[r, sl] * sv
            return 0
        lax.fori_loop(0, 128, _scale, 0)
        pltpu.sync_copy(gbuf, eacc_o.at[cid, pl.ds(pl.multiple_of(lo, 64), 128)])


def _run_sc_ent(ent_src, ent_dst, ent_tab):
    scratch = [
        pltpu.VMEM((10, 1, 128), I32),
        pltpu.VMEM((20, 1, 128), I32),
        pltpu.VMEM((1, 128), F32),
        pltpu.VMEM((128, 128), F32),
        pltpu.VMEM((640,), F32),
        pltpu.VMEM((640,), F32),
        pltpu.SemaphoreType.DMA,
        pltpu.VMEM_SHARED((NP, EMB), F32),
        pltpu.VMEM_SHARED((NP,), F32),
    ]
    fn = pl.kernel(_sc_ent, out_type=jax.ShapeDtypeStruct((NC, NP, EMB), F32),
                   mesh=_mesh(), scratch_types=scratch,
                   compiler_params=_SC_PARAMS)
    return fn(ent_src, ent_dst, ent_tab)


def _sc_relw(rel_src, rel_dst, tim_src, tim_dst,
             trr, trc, ttr, ttc, tval, rel_tab, tim_tab,
             racc_o, tacc_o, wr_o, wt_o,
             ridxa, ridxb, tidxa, tidxb,
             rr_b, rc_b, tr_b, tc_b, tv_b, tflat, ones_b,
             gbuf, zflat, dbuf, rcps, sem,
             s_racc, s_rcnt, s_tacc, s_tcnt, s_wr, s_wt):
    cid = lax.axis_index("c")
    sid = lax.axis_index("s")
    wid = sid * NC + cid

    _zero_gbuf(gbuf)
    _zero_1d(zflat, 2048)
    for i in range(8):
        ones_b[0, pl.ds(i * L, L)] = jnp.ones((L,), F32)
    pltpu.sync_copy(gbuf.at[pl.ds(0, 32)], s_racc.at[pl.ds(sid * 32, 32)])
    pltpu.sync_copy(gbuf.at[pl.ds(0, 32)], s_tacc.at[pl.ds(sid * 32, 32)])
    pltpu.sync_copy(zflat.at[pl.ds(0, 32)], s_rcnt.at[pl.ds(sid * 32, 32)])
    pltpu.sync_copy(zflat.at[pl.ds(0, 32)], s_tcnt.at[pl.ds(sid * 32, 32)])
    for k in range(8):
        pltpu.sync_copy(zflat, s_wr.at[pl.ds(sid * 16384 + k * 2048, 2048)])
        pltpu.sync_copy(zflat, s_wt.at[pl.ds(sid * 16384 + k * 2048, 2048)])
    plsc.subcore_barrier()

    pltpu.sync_copy(rel_dst.at[pl.ds(sid * 2, 2)], ridxb)
    pltpu.sync_copy(rel_src.at[pl.ds(wid, 1)], ridxa)
    pltpu.sync_copy(tim_dst.at[pl.ds(sid * 2, 2)], tidxb)
    pltpu.sync_copy(tim_src.at[pl.ds(wid, 1)], tidxa)
    for j in range(2):
        pltpu.sync_copy(ones_b.at[0], s_rcnt.at[ridxb.at[j, 0]], add=True)
        pltpu.sync_copy(ones_b.at[0], s_tcnt.at[tidxb.at[j, 0]], add=True)
    pltpu.async_copy(rel_tab.at[ridxa.at[0, 0]], gbuf, sem).wait()
    pltpu.sync_copy(gbuf, s_racc.at[ridxb.at[cid, 0]], add=True)
    pltpu.async_copy(tim_tab.at[tidxa.at[0, 0]], gbuf, sem).wait()
    pltpu.sync_copy(gbuf, s_tacc.at[tidxb.at[cid, 0]], add=True)

    pltpu.sync_copy(trr.at[pl.ds(wid * 40, 40)], rr_b)
    pltpu.sync_copy(trc.at[pl.ds(wid * 40, 40)], rc_b)
    pltpu.sync_copy(ttr.at[pl.ds(wid * 40, 40)], tr_b)
    pltpu.sync_copy(ttc.at[pl.ds(wid * 40, 40)], tc_b)
    pltpu.sync_copy(tval.at[pl.ds(wid * 40, 40)], tv_b)

    def _tri(j, _):
        for jj in range(8):
            sl = pl.ds(jj * L, L)
            tflat[0, sl] = rr_b[j, 0, sl] * 512 + rc_b[j, 0, sl]
        pltpu.sync_copy(tv_b.at[j, 0], s_wr.at[tflat.at[0]], add=True)
        for jj in range(8):
            sl = pl.ds(jj * L, L)
            tflat[0, sl] = tr_b[j, 0, sl] * 512 + tc_b[j, 0, sl]
        pltpu.sync_copy(tv_b.at[j, 0], s_wt.at[tflat.at[0]], add=True)
        return 0
    lax.fori_loop(0, 40, _tri, 0)

    plsc.subcore_barrier()

    for (cnt_ref, acc_ref, out_ref) in ((s_rcnt, s_racc, racc_o),
                                        (s_tcnt, s_tacc, tacc_o)):
        pltpu.sync_copy(cnt_ref.at[pl.ds(sid * 32, 32)], rcps.at[pl.ds(0, 32)])
        for i in range(2):
            sl = pl.ds(i * L, L)
            rcps[sl] = 1.0 / jnp.maximum(rcps[sl], 1.0)
        pltpu.sync_copy(acc_ref.at[pl.ds(sid * 32, 32)], dbuf)

        def _scale_s(r, _):
            sv = plsc.load_gather(rcps, [jnp.full((L,), r, I32)])
            for f in range(8):
                sl = pl.ds(f * L, L)
                dbuf[r, sl] = dbuf[r, sl] * sv
            return 0
        lax.fori_loop(0, 32, _scale_s, 0)
        pltpu.sync_copy(dbuf,
                        out_ref.at[cid, pl.ds(pl.multiple_of(sid * 32, 8), 32)])

    for k in range(8):
        o = sid * 16384 + k * 2048
        pltpu.sync_copy(s_wr.at[pl.ds(o, 2048)], zflat)
        pltpu.sync_copy(zflat, wr_o.at[cid, pl.ds(pl.multiple_of(o, 128), 2048)])
        pltpu.sync_copy(s_wt.at[pl.ds(o, 2048)], zflat)
        pltpu.sync_copy(zflat, wt_o.at[cid, pl.ds(pl.multiple_of(o, 128), 2048)])


def _run_sc_relw(rel_src, rel_dst, tim_src, tim_dst,
                 trr, trc, ttr, ttc, tval, rel_tab, tim_tab):
    out_type = (
        jax.ShapeDtypeStruct((NC, RP, EMB), F32),
        jax.ShapeDtypeStruct((NC, RP, EMB), F32),
        jax.ShapeDtypeStruct((NC, WF), F32),
        jax.ShapeDtypeStruct((NC, WF), F32),
    )
    scratch = [
        pltpu.VMEM((1, 1, 128), I32),
        pltpu.VMEM((2, 1, 128), I32),
        pltpu.VMEM((1, 1, 128), I32),
        pltpu.VMEM((2, 1, 128), I32),
        pltpu.VMEM((40, 1, 128), I32),
        pltpu.VMEM((40, 1, 128), I32),
        pltpu.VMEM((40, 1, 128), I32),
        pltpu.VMEM((40, 1, 128), I32),
        pltpu.VMEM((40, 1, 128), F32),
        pltpu.VMEM((1, 128), I32),
        pltpu.VMEM((1, 128), F32),
        pltpu.VMEM((128, 128), F32),
        pltpu.VMEM((2048,), F32),
        pltpu.VMEM((32, 128), F32),
        pltpu.VMEM((64,), F32),
        pltpu.SemaphoreType.DMA,
        pltpu.VMEM_SHARED((RP, EMB), F32),
        pltpu.VMEM_SHARED((RP,), F32),
        pltpu.VMEM_SHARED((RP, EMB), F32),
        pltpu.VMEM_SHARED((RP,), F32),
        pltpu.VMEM_SHARED((WF,), F32),
        pltpu.VMEM_SHARED((WF,), F32),
    ]
    fn = pl.kernel(_sc_relw, out_type=out_type, mesh=_mesh(),
                   scratch_types=scratch, compiler_params=_SC_PARAMS)
    return fn(rel_src, rel_dst, tim_src, tim_dst,
              trr, trc, ttr, ttc, tval, rel_tab, tim_tab)


def _tc_small(racc, tacc, wr2, wt2, wcr, wct,
              cr_o, ct_o, u_o, mcr_o, mct_o):
    rel_emb = racc[0] + racc[1]
    tim_emb = tacc[0] + tacc[1]
    w_r = wr2[0] + wr2[1]
    w_t = wt2[0] + wt2[1]
    rels = jnp.dot(w_r, rel_emb, preferred_element_type=F32)
    times = jnp.dot(w_t, tim_emb, preferred_element_type=F32)
    u_o[...] = times
    rn = rels / jnp.maximum(
        jnp.sqrt(jnp.sum(rels * rels, axis=1, keepdims=True)), 1e-12)
    tn = times / jnp.maximum(
        jnp.sqrt(jnp.sum(times * times, axis=1, keepdims=True)), 1e-12)
    cr = jnp.dot(rn, wcr[...], preferred_element_type=F32)
    ct = jnp.dot(tn, wct[...], preferred_element_type=F32)
    cr_o[...] = cr
    ct_o[...] = ct
    mcr_o[...] = jnp.broadcast_to(jnp.max(cr), (1, 128))
    mct_o[...] = jnp.broadcast_to(jnp.max(ct), (1, 128))


def _run_tc_small(racc, tacc, wr2, wt2, wcr, wct):
    return pl.pallas_call(
        _tc_small,
        out_shape=(
            jax.ShapeDtypeStruct((RP, 1), F32),
            jax.ShapeDtypeStruct((RP, 1), F32),
            jax.ShapeDtypeStruct((RP, EMB), F32),
            jax.ShapeDtypeStruct((1, 128), F32),
            jax.ShapeDtypeStruct((1, 128), F32),
        ),
    )(racc, tacc, wr2, wt2, wcr, wct)


def _tc_proj(e0, e1, wcols, ent_o, relu_o, pq_o, mx_o):
    i = pl.program_id(0)
    ent = e0[...] + e1[...]
    ent_o[...] = ent
    relu_o[...] = jnp.maximum(ent, 0.0)
    pq = jnp.dot(ent, wcols[...], preferred_element_type=F32)
    pq_o[...] = pq
    bm = jnp.max(pq, axis=0, keepdims=True)

    @pl.when(i == 0)
    def _():
        mx_o[...] = jnp.full((8, 8), -3e38, F32)
    mx_o[0:1, :] = jnp.maximum(mx_o[0:1, :], bm)


def _run_tc_proj(e0, e1, wcols):
    nb = NP // 1024
    return pl.pallas_call(
        _tc_proj,
        grid=(nb,),
        in_specs=[
            pl.BlockSpec((1024, EMB), lambda i: (i, 0)),
            pl.BlockSpec((1024, EMB), lambda i: (i, 0)),
            pl.BlockSpec((EMB, 8), lambda i: (0, 0)),
        ],
        out_specs=(
            pl.BlockSpec((1024, EMB), lambda i: (i, 0)),
            pl.BlockSpec((1024, EMB), lambda i: (i, 0)),
            pl.BlockSpec((1024, 8), lambda i: (i, 0)),
            pl.BlockSpec((8, 8), lambda i: (0, 0)),
        ),
        out_shape=(
            jax.ShapeDtypeStruct((NP, EMB), F32),
            jax.ShapeDtypeStruct((NP, EMB), F32),
            jax.ShapeDtypeStruct((NP, 8), F32),
            jax.ShapeDtypeStruct((8, 8), F32),
        ),
    )(e0, e1, wcols)


def _sc_soft(rows3d, cols3d, pqt, crow, biasv, lane,
             a_o,
             prb, qrb, ptb, qtb, crb, ctb, bb, lb,
             rbuf, cbuf, sbr, sbt, zflat,
             s_dr, s_dt):
    cid = lax.axis_index("c")
    sid = lax.axis_index("s")
    wid = sid * NC + cid

    _zero_1d(zflat, 640)
    pltpu.sync_copy(zflat, s_dr.at[pl.ds(sid * 640, 640)])
    pltpu.sync_copy(zflat, s_dt.at[pl.ds(sid * 640, 640)])

    pltpu.sync_copy(pqt.at[0, 0], prb)
    pltpu.sync_copy(pqt.at[1, 0], qrb)
    pltpu.sync_copy(pqt.at[2, 0], ptb)
    pltpu.sync_copy(pqt.at[3, 0], qtb)
    pltpu.sync_copy(crow.at[0, 0], crb)
    pltpu.sync_copy(crow.at[1, 0], ctb)
    pltpu.sync_copy(biasv, bb)
    pltpu.sync_copy(lane, lb)
    pltpu.sync_copy(rows3d.at[pl.ds(sid * 80, 80)], rbuf)
    pltpu.sync_copy(cols3d.at[pl.ds(sid * 80, 80)], cbuf)
    plsc.subcore_barrier()

    bias_r = bb[pl.ds(0, L)]
    bias_t = bb[pl.ds(L, L)]
    lanev = lb[...]

    def _p1(j, _):
        for jj in range(8):
            sl = pl.ds(jj * L, L)
            r16 = rbuf[j, 0, sl]
            c16 = cbuf[j, 0, sl]
            e16 = sid * 10240 + j * 128 + jj * L + lanev
            cidx = jnp.minimum(e16, 511)
            xr = (plsc.load_gather(prb, [r16]) + plsc.load_gather(qrb, [c16])
                  + plsc.load_gather(crb, [cidx]) + bias_r)
            xt = (plsc.load_gather(ptb, [r16]) + plsc.load_gather(qtb, [c16])
                  + plsc.load_gather(ctb, [cidx]) + bias_t)
            sbr[j, 0, sl] = jnp.exp(xr)
            sbt[j, 0, sl] = jnp.exp(xt)
        pltpu.sync_copy(sbr.at[j, 0], s_dr.at[rbuf.at[j, 0]], add=True)
        pltpu.sync_copy(sbt.at[j, 0], s_dt.at[rbuf.at[j, 0]], add=True)
        return 0
    lax.fori_loop(0, 80, _p1, 0)

    plsc.subcore_barrier()

    pltpu.sync_copy(s_dr, prb)
    pltpu.sync_copy(s_dt, qrb)

    def _p2(j2, _):
        j = cid * 40 + j2
        for jj in range(8):
            sl = pl.ds(jj * L, L)
            r16 = rbuf[j, 0, sl]
            dr = plsc.load_gather(prb, [r16])
            dt = plsc.load_gather(qrb, [r16])
            sbr[j, 0, sl] = (sbr[j, 0, sl] / jnp.maximum(dr, 1e-12)
                             + sbt[j, 0, sl] / jnp.maximum(dt, 1e-12))
        pltpu.sync_copy(sbr.at[j, 0],
                        a_o.at[sid * 80 + j, 0])
        return 0
    lax.fori_loop(0, 40, _p2, 0)


def _run_sc_soft(rows3d, cols3d, pqt, crow, biasv, lane):
    scratch = [
        pltpu.VMEM((NP,), F32),
        pltpu.VMEM((NP,), F32),
        pltpu.VMEM((NP,), F32),
        pltpu.VMEM((NP,), F32),
        pltpu.VMEM((RP,), F32),
        pltpu.VMEM((RP,), F32),
        pltpu.VMEM((32,), F32),
        pltpu.VMEM((16,), I32),
        pltpu.VMEM((80, 1, 128), I32),
        pltpu.VMEM((80, 1, 128), I32),
        pltpu.VMEM((80, 1, 128), F32),
        pltpu.VMEM((80, 1, 128), F32),
        pltpu.VMEM((640,), F32),
        pltpu.VMEM_SHARED((NP,), F32),
        pltpu.VMEM_SHARED((NP,), F32),
    ]
    fn = pl.kernel(_sc_soft,
                   out_type=jax.ShapeDtypeStruct((TP // 128, 1, 128), F32),
                   mesh=_mesh(), scratch_types=scratch,
                   compiler_params=_SC_PARAMS)
    return fn(rows3d, cols3d, pqt, crow, biasv, lane)


def _sc_agg(rows3d, cols3d, a3, ent_emb,
            acc_o,
            rbuf, cbuf, abuf, gbuf, gbuf2, sem, sem2,
            s_acc):
    cid = lax.axis_index("c")
    sid = lax.axis_index("s")
    wid = sid * NC + cid

    _zero_gbuf(gbuf)
    for k in range(5):
        pltpu.sync_copy(gbuf, s_acc.at[pl.ds(sid * 640 + k * 128, 128)])
    pltpu.sync_copy(rows3d.at[pl.ds(wid * 40, 40)], rbuf)
    pltpu.sync_copy(cols3d.at[pl.ds(wid * 40, 40)], cbuf)
    pltpu.sync_copy(a3.at[pl.ds(wid * 40, 40)], abuf)
    plsc.subcore_barrier()

    bufs = (gbuf, gbuf2)
    sems = (sem, sem2)
    cp = {0: pltpu.async_copy(ent_emb.at[cbuf.at[0, 0]], bufs[0], sems[0])}
    for j in range(40):
        b = bufs[j % 2]
        if j + 1 < 40:
            cp[(j + 1) % 2] = pltpu.async_copy(
                ent_emb.at[cbuf.at[j + 1, 0]], bufs[(j + 1) % 2],
                sems[(j + 1) % 2])
        cp[j % 2].wait()

        def _scale(r, _):
            jf = jnp.full((L,), j, I32)
            av = plsc.load_gather(abuf, [jf, jnp.minimum(jf, 0),
                                         jnp.full((L,), r, I32)])
            for f in range(8):
                sl = pl.ds(f * L, L)
                b[r, sl] = b[r, sl] * av
            return 0
        lax.fori_loop(0, 128, _scale, 0)
        pltpu.sync_copy(b, s_acc.at[rbuf.at[j, 0]], add=True)

    plsc.subcore_barrier()
    for k in range(5):
        lo = sid * 640 + k * 128
        pltpu.sync_copy(s_acc.at[pl.ds(lo, 128)], gbuf)
        pltpu.sync_copy(gbuf, acc_o.at[cid, pl.ds(pl.multiple_of(lo, 64), 128)])


def _run_sc_agg(rows3d, cols3d, a3, ent_emb):
    scratch = [
        pltpu.VMEM((40, 1, 128), I32),
        pltpu.VMEM((40, 1, 128), I32),
        pltpu.VMEM((40, 1, 128), F32),
        pltpu.VMEM((128, 128), F32),
        pltpu.VMEM((128, 128), F32),
        pltpu.SemaphoreType.DMA,
        pltpu.SemaphoreType.DMA,
        pltpu.VMEM_SHARED((NP, EMB), F32),
    ]
    fn = pl.kernel(_sc_agg,
                   out_type=jax.ShapeDtypeStruct((NC, NP, EMB), F32),
                   mesh=_mesh(), scratch_types=scratch,
                   compiler_params=_SC_PARAMS)
    return fn(rows3d, cols3d, a3, ent_emb)


def _tc_out(n0, n1, relu_ent, u, rows512, out_o):
    i = pl.program_id(0)
    feat = jnp.maximum(n0[...] + n1[...], 0.0)
    node_ids = lax.broadcasted_iota(I32, (RP, 1000), 1) + i * 1000
    onehot = jnp.where(node_ids == rows512[:, 0:1], 1.0, 0.0)
    tenc = jnp.maximum(
        lax.dot_general(onehot, u[...], (((0,), (0,)), ((), ())),
                        preferred_element_type=F32), 0.0)
    out_o[:, 0:128] = relu_ent[...]
    out_o[:, 128:256] = feat
    out_o[:, 256:384] = feat
    out_o[:, 384:512] = tenc


def _run_tc_out(n0, n1, relu_ent, u, rows512):
    return pl.pallas_call(
        _tc_out,
        grid=(10,),
        in_specs=[
            pl.BlockSpec((1000, EMB), lambda i: (i, 0)),
            pl.BlockSpec((1000, EMB), lambda i: (i, 0)),
            pl.BlockSpec((1000, EMB), lambda i: (i, 0)),
            pl.BlockSpec((RP, EMB), lambda i: (0, 0)),
            pl.BlockSpec((RP, 8), lambda i: (0, 0)),
        ],
        out_specs=pl.BlockSpec((1000, 4 * EMB), lambda i: (i, 0)),
        out_shape=jax.ShapeDtypeStruct((NODE, 4 * EMB), F32),
    )(n0, n1, relu_ent, u, rows512)


def kernel(r_index_sq, r_val_sq, t_index_sq, adj_idx, ent_sparse, rel_sparse,
           time_sparse, ent_table, rel_table, time_table, attend_rel_w,
           attend_rel_b, attend_time_w, attend_time_b):
    def pad1(x, n, fill):
        x = x.astype(I32)
        return jnp.concatenate([x, fill.astype(I32)]) if n > x.shape[0] else x

    dump16 = lambda m: NODE + (jnp.arange(m, dtype=I32) % 16)

    ent_src = pad1(ent_sparse[1], EP, jnp.full((EP - 4 * NODE,), NODE + 8))
    ent_dst = pad1(ent_sparse[0], EP, dump16(EP - 4 * NODE))
    rel_src = pad1(rel_sparse[1], SP, jnp.full((SP - 8 * REL,), NODE + 8))
    rel_dst = pad1(rel_sparse[0], SP,
                   504 + (jnp.arange(SP - 8 * REL, dtype=I32) % 8))
    tim_src = pad1(time_sparse[1], SP, jnp.full((SP - 8 * REL,), NODE + 8))
    tim_dst = pad1(time_sparse[0], SP,
                   504 + (jnp.arange(SP - 8 * REL, dtype=I32) % 8))

    npad = TP - TRIPLE
    trr = pad1(r_index_sq[:, 0], TP, jnp.full((npad,), 500))
    trc = pad1(r_index_sq[:, 1], TP, jnp.arange(npad, dtype=I32) % 500)
    ttr = pad1(t_index_sq[:, 0], TP, jnp.full((npad,), 500))
    ttc = pad1(t_index_sq[:, 1], TP, jnp.arange(npad, dtype=I32) % 500)
    tval = jnp.concatenate([r_val_sq, jnp.zeros((npad,), F32)])

    rows = pad1(adj_idx[0], TP, dump16(npad))
    cols = pad1(adj_idx[1], TP, jnp.full((npad,), NODE + 8))

    pad_tab = lambda t: jnp.concatenate(
        [t, jnp.zeros((NP - NODE, EMB), F32)], axis=0)
    ent_tab = pad_tab(ent_table)
    rel_tab = pad_tab(rel_table)
    tim_tab = pad_tab(time_table)

    r3 = lambda x, n: x.reshape(n, 1, 128)

    eacc = _run_sc_ent(r3(ent_src, EP // 128), r3(ent_dst, EP // 128),
                       ent_tab)
    racc, tacc, wr, wt = _run_sc_relw(
        r3(rel_src, SP // 128), r3(rel_dst, SP // 128),
        r3(tim_src, SP // 128), r3(tim_dst, SP // 128),
        r3(trr, TP // 128), r3(trc, TP // 128),
        r3(ttr, TP // 128), r3(ttc, TP // 128),
        r3(tval, TP // 128), rel_tab, tim_tab)

    wcr = attend_rel_w[0, 256:384].reshape(EMB, 1)
    wct = attend_time_w[0, 256:384].reshape(EMB, 1)
    cr, ct, u, mcr, mct = _run_tc_small(
        racc, tacc, wr.reshape(NC, 512, 512), wt.reshape(NC, 512, 512),
        wcr, wct)

    wcols = jnp.stack([attend_rel_w[0, 0:128], attend_rel_w[0, 128:256],
                       attend_time_w[0, 0:128], attend_time_w[0, 128:256],
                       jnp.zeros((EMB,), F32), jnp.zeros((EMB,), F32),
                       jnp.zeros((EMB,), F32), jnp.zeros((EMB,), F32)],
                      axis=1)
    ent_emb, relu_ent, pq, mx = _run_tc_proj(eacc[0], eacc[1], wcols)

    pqt = pq.T[0:4].reshape(4, 1, NP)
    crow = jnp.stack([cr.reshape(RP), ct.reshape(RP)]).reshape(2, 1, RP)
    bias_r = attend_rel_b[0] - mx[0, 0] - mx[0, 1] - mcr[0, 0]
    bias_t = attend_time_b[0] - mx[0, 2] - mx[0, 3] - mct[0, 0]
    biasv = jnp.concatenate([jnp.full((16,), bias_r, F32),
                             jnp.full((16,), bias_t, F32)])

    a3 = _run_sc_soft(r3(rows, TP // 128), r3(cols, TP // 128),
                      pqt, crow, biasv, jnp.arange(16, dtype=I32))
    acc = _run_sc_agg(r3(rows, TP // 128), r3(cols, TP // 128), a3, ent_emb)

    rows512 = jnp.concatenate(
        [adj_idx[0][:500].astype(I32), jnp.full((12,), 1 << 28, I32)])
    rows512 = jnp.broadcast_to(rows512.reshape(RP, 1), (RP, 8))
    return _run_tc_out(acc[0], acc[1], relu_ent, u, rows512)
